# baseline (speedup 1.0000x reference)
import functools

import jax
import jax.numpy as jnp
from jax import lax
from jax.experimental import pallas as pl
from jax.experimental.pallas import tpu as pltpu
from jax.experimental.pallas import tpu_sc as plsc

N = 10000
E = 160000
D = 256
DH = 128
NLAYERS = 3
NSUB = 16
CHUNK = 128
NCHUNK = 80
HCHUNK = NCHUNK // 2
EPW = NCHUNK * CHUNK
EPAD = NSUB * EPW
NROWS_ACC = N + 8
DUMMY_DST = N
SLAB = 632
SLAB_LAST = N - (NSUB - 1) * SLAB
BLK = 2000
NB = N // BLK
EPS = 1e-5


def _sc_body(h_hbm, src_hbm, dst_hbm, z_hbm, acc, sidx, didx, rows0, rows1,
             semA, semB):
  c = lax.axis_index("c")
  s = lax.axis_index("s")


  def slab_copy(src_ref, dst_ref):
    r0 = s * SLAB

    @pl.when(s < NSUB - 1)
    def _():
      pltpu.sync_copy(src_ref.at[pl.ds(r0, SLAB)],
                      dst_ref.at[pl.ds(r0, SLAB)])

    @pl.when(s == NSUB - 1)
    def _():
      pltpu.sync_copy(src_ref.at[pl.ds((NSUB - 1) * SLAB, SLAB_LAST)],
                      dst_ref.at[pl.ds((NSUB - 1) * SLAB, SLAB_LAST)])

  def run_half(h_half, z_half):
    slab_copy(h_half, acc)
    plsc.subcore_barrier()

    for half in range(2):
      pltpu.sync_copy(src_hbm.at[s, pl.ds(half * HCHUNK, HCHUNK)], sidx)
      pltpu.sync_copy(dst_hbm.at[s, pl.ds(half * HCHUNK, HCHUNK)], didx)
      pltpu.sync_copy(h_half.at[sidx.at[0]], rows0)

      @pl.loop(0, HCHUNK - 2, step=2)
      def _(j):
        sA = pltpu.async_copy(rows0, acc.at[didx.at[j]], semA, add=True)
        pltpu.sync_copy(h_half.at[sidx.at[j + 1]], rows1)
        sA.wait()
        sB = pltpu.async_copy(rows1, acc.at[didx.at[j + 1]], semB, add=True)
        pltpu.sync_copy(h_half.at[sidx.at[j + 2]], rows0)
        sB.wait()

      sA = pltpu.async_copy(rows0, acc.at[didx.at[HCHUNK - 2]], semA,
                            add=True)
      pltpu.sync_copy(h_half.at[sidx.at[HCHUNK - 1]], rows1)
      sA.wait()
      pltpu.sync_copy(rows1, acc.at[didx.at[HCHUNK - 1]], add=True)

    plsc.subcore_barrier()
    slab_copy(acc, z_half)

  @pl.when(c == 0)
  def _():
    run_half(h_hbm.at[0], z_hbm.at[0])

  @pl.when(c == 1)
  def _():
    run_half(h_hbm.at[1], z_hbm.at[1])


def _sc_aggregate(h2, src_r, dst_r):
  mesh = plsc.VectorSubcoreMesh(core_axis_name="c", subcore_axis_name="s")
  kern = pl.kernel(
      _sc_body,
      out_type=jax.ShapeDtypeStruct((2, N, DH), jnp.float32),
      mesh=mesh,
      scratch_types=[
          pltpu.VMEM_SHARED((NROWS_ACC, DH), jnp.float32),
          pltpu.VMEM((HCHUNK, CHUNK), jnp.int32),
          pltpu.VMEM((HCHUNK, CHUNK), jnp.int32),
          pltpu.VMEM((CHUNK, DH), jnp.float32),
          pltpu.VMEM((CHUNK, DH), jnp.float32),
          pltpu.SemaphoreType.DMA,
          pltpu.SemaphoreType.DMA,
      ],
  )
  return kern(h2, src_r, dst_r)


def _mlp_body(last, z, w1, b1, g1, bt1, w2, b2, g2, bt2, out, u):
  w1m = w1[...]
  w2m = w2[...]
  b1v = b1[...]
  b2v = b2[...]
  zero = jnp.zeros((1, D), jnp.float32)

  def p1(i, carry):
    s0, s1 = carry
    zL = z[0, pl.ds(i * BLK, BLK), :]
    zR = z[1, pl.ds(i * BLK, BLK), :]
    ub = (jnp.dot(zL, w1m[:DH, :], preferred_element_type=jnp.float32)
          + jnp.dot(zR, w1m[DH:, :], preferred_element_type=jnp.float32)
          + b1v)
    u[pl.ds(i * BLK, BLK), :] = ub
    return (s0 + jnp.sum(ub, axis=0, keepdims=True),
            s1 + jnp.sum(ub * ub, axis=0, keepdims=True))

  s0, s1 = lax.fori_loop(0, NB, p1, (zero, zero))
  m1 = s0 / N
  v1 = s1 / N - m1 * m1
  sc1 = g1[...] * lax.rsqrt(v1 + EPS)
  sh1 = bt1[...] - m1 * sc1

  def p2(i, carry):
    s0, s1 = carry
    ub = u[pl.ds(i * BLK, BLK), :]
    r = jnp.maximum(ub * sc1 + sh1, 0.0)
    sb = jnp.dot(r, w2m, preferred_element_type=jnp.float32) + b2v
    u[pl.ds(i * BLK, BLK), :] = sb
    return (s0 + jnp.sum(sb, axis=0, keepdims=True),
            s1 + jnp.sum(sb * sb, axis=0, keepdims=True))

  s0, s1 = lax.fori_loop(0, NB, p2, (zero, zero))
  m2 = s0 / N
  v2 = s1 / N - m2 * m2
  sc2 = g2[...] * lax.rsqrt(v2 + EPS)
  sh2 = bt2[...] - m2 * sc2

  def p3(i, _):
    sb = u[pl.ds(i * BLK, BLK), :]
    h = sb * sc2 + sh2
    if not last:
      h = jnp.maximum(h, 0.0)
      out[0, pl.ds(i * BLK, BLK), :] = h[:, :DH]
      out[1, pl.ds(i * BLK, BLK), :] = h[:, DH:]
    else:
      out[pl.ds(i * BLK, BLK), :] = h
    return 0

  lax.fori_loop(0, NB, p3, 0)


def _mlp_layer(z2, w1, b1, g1, bt1, w2, b2, g2, bt2, last):
  out_shape = (jax.ShapeDtypeStruct((N, D), jnp.float32) if last
               else jax.ShapeDtypeStruct((2, N, DH), jnp.float32))
  return pl.pallas_call(
      functools.partial(_mlp_body, last),
      out_shape=out_shape,
      scratch_shapes=[pltpu.VMEM((N, D), jnp.float32)],
  )(z2, w1, b1, g1, bt1, w2, b2, g2, bt2)


def kernel(x, W1, b1, g1, bt1, W2, b2, g2, bt2, edge_index, batch):
  src = edge_index[0].astype(jnp.int32)
  dst = edge_index[1].astype(jnp.int32)
  pad = EPAD - E
  src_r = jnp.concatenate(
      [src, jnp.zeros((pad,), jnp.int32)]).reshape(NSUB, NCHUNK, CHUNK)
  dst_r = jnp.concatenate(
      [dst, jnp.full((pad,), DUMMY_DST, jnp.int32)]).reshape(
          NSUB, NCHUNK, CHUNK)

  h2 = x.reshape(N, 2, DH).transpose(1, 0, 2)
  for l in range(NLAYERS):
    z2 = _sc_aggregate(h2, src_r, dst_r)
    last = l == NLAYERS - 1
    h2 = _mlp_layer(
        z2,
        W1[l], b1[l].reshape(1, D), g1[l].reshape(1, D),
        bt1[l].reshape(1, D),
        W2[l], b2[l].reshape(1, D), g2[l].reshape(1, D),
        bt2[l].reshape(1, D),
        last)
  return (h2, batch)


# speedup vs baseline: 1.6553x; 1.6553x over previous
import functools

import jax
import jax.numpy as jnp
from jax import lax
from jax.experimental import pallas as pl
from jax.experimental.pallas import tpu as pltpu
from jax.experimental.pallas import tpu_sc as plsc

N = 10000
E = 160000
D = 256
DH = 128
NLAYERS = 3
NSUB = 16
CHUNK = 128
NCHUNK = 78
NTAIL = 2
EPW = NCHUNK * CHUNK
EPAD = NSUB * EPW
NROWS_ACC = N + 8
DUMMY_DST = N
SLAB = 632
SLAB_LAST = N - (NSUB - 1) * SLAB
BLK = 2000
NB = N // BLK
EPS = 1e-5


def _sc_body(h_hbm, src_hbm, dst_hbm, st_hbm, dt_hbm, z_hbm, acc, sidx, didx,
             tidx_s, tidx_d, rows0, sem0):
  c = lax.axis_index("c")
  s = lax.axis_index("s")


  def slab_copy(src_ref, dst_ref):
    r0 = s * SLAB

    @pl.when(s < NSUB - 1)
    def _():
      pltpu.sync_copy(src_ref.at[pl.ds(r0, SLAB)],
                      dst_ref.at[pl.ds(r0, SLAB)])

    @pl.when(s == NSUB - 1)
    def _():
      pltpu.sync_copy(src_ref.at[pl.ds((NSUB - 1) * SLAB, SLAB_LAST)],
                      dst_ref.at[pl.ds((NSUB - 1) * SLAB, SLAB_LAST)])

  def run_half(h_half, z_half):
    slab_copy(h_half, acc)
    plsc.subcore_barrier()

    pltpu.sync_copy(src_hbm.at[s], sidx)
    pltpu.sync_copy(dst_hbm.at[s], didx)

    @pl.loop(0, NCHUNK)
    def _(j):
      pltpu.sync_copy(h_half.at[sidx.at[j]], rows0)
      pltpu.sync_copy(rows0, acc.at[didx.at[j]], add=True)

    @pl.when(s < NTAIL)
    def _():
      pltpu.sync_copy(st_hbm.at[s], tidx_s)
      pltpu.sync_copy(dt_hbm.at[s], tidx_d)
      pltpu.sync_copy(h_half.at[tidx_s.at[0]], rows0)
      pltpu.sync_copy(rows0, acc.at[tidx_d.at[0]], add=True)

    plsc.subcore_barrier()
    slab_copy(acc, z_half)

  @pl.when(c == 0)
  def _():
    run_half(h_hbm.at[0], z_hbm.at[0])

  @pl.when(c == 1)
  def _():
    run_half(h_hbm.at[1], z_hbm.at[1])


def _sc_aggregate(h2, src_r, dst_r, src_t, dst_t):
  mesh = plsc.VectorSubcoreMesh(core_axis_name="c", subcore_axis_name="s")
  kern = pl.kernel(
      _sc_body,
      out_type=jax.ShapeDtypeStruct((2, N, DH), jnp.float32),
      mesh=mesh,
      scratch_types=[
          pltpu.VMEM_SHARED((NROWS_ACC, DH), jnp.float32),
          pltpu.VMEM((NCHUNK, CHUNK), jnp.int32),
          pltpu.VMEM((NCHUNK, CHUNK), jnp.int32),
          pltpu.VMEM((1, CHUNK), jnp.int32),
          pltpu.VMEM((1, CHUNK), jnp.int32),
          pltpu.VMEM((CHUNK, DH), jnp.float32),
          pltpu.SemaphoreType.DMA,
      ],
  )
  return kern(h2, src_r, dst_r, src_t, dst_t)


def _mlp_body(last, z, w1, b1, g1, bt1, w2, b2, g2, bt2, out, u):
  w1m = w1[...]
  w2m = w2[...]
  b1v = b1[...]
  b2v = b2[...]
  zero = jnp.zeros((1, D), jnp.float32)

  def p1(i, carry):
    s0, s1 = carry
    zL = z[0, pl.ds(i * BLK, BLK), :]
    zR = z[1, pl.ds(i * BLK, BLK), :]
    ub = (jnp.dot(zL, w1m[:DH, :], preferred_element_type=jnp.float32)
          + jnp.dot(zR, w1m[DH:, :], preferred_element_type=jnp.float32)
          + b1v)
    u[pl.ds(i * BLK, BLK), :] = ub
    return (s0 + jnp.sum(ub, axis=0, keepdims=True),
            s1 + jnp.sum(ub * ub, axis=0, keepdims=True))

  s0, s1 = lax.fori_loop(0, NB, p1, (zero, zero))
  m1 = s0 / N
  v1 = s1 / N - m1 * m1
  sc1 = g1[...] * lax.rsqrt(v1 + EPS)
  sh1 = bt1[...] - m1 * sc1

  def p2(i, carry):
    s0, s1 = carry
    ub = u[pl.ds(i * BLK, BLK), :]
    r = jnp.maximum(ub * sc1 + sh1, 0.0)
    sb = jnp.dot(r, w2m, preferred_element_type=jnp.float32) + b2v
    u[pl.ds(i * BLK, BLK), :] = sb
    return (s0 + jnp.sum(sb, axis=0, keepdims=True),
            s1 + jnp.sum(sb * sb, axis=0, keepdims=True))

  s0, s1 = lax.fori_loop(0, NB, p2, (zero, zero))
  m2 = s0 / N
  v2 = s1 / N - m2 * m2
  sc2 = g2[...] * lax.rsqrt(v2 + EPS)
  sh2 = bt2[...] - m2 * sc2

  def p3(i, _):
    sb = u[pl.ds(i * BLK, BLK), :]
    h = sb * sc2 + sh2
    if not last:
      h = jnp.maximum(h, 0.0)
      out[0, pl.ds(i * BLK, BLK), :] = h[:, :DH]
      out[1, pl.ds(i * BLK, BLK), :] = h[:, DH:]
    else:
      out[pl.ds(i * BLK, BLK), :] = h
    return 0

  lax.fori_loop(0, NB, p3, 0)


def _mlp_layer(z2, w1, b1, g1, bt1, w2, b2, g2, bt2, last):
  out_shape = (jax.ShapeDtypeStruct((N, D), jnp.float32) if last
               else jax.ShapeDtypeStruct((2, N, DH), jnp.float32))
  return pl.pallas_call(
      functools.partial(_mlp_body, last),
      out_shape=out_shape,
      scratch_shapes=[pltpu.VMEM((N, D), jnp.float32)],
  )(z2, w1, b1, g1, bt1, w2, b2, g2, bt2)


def kernel(x, W1, b1, g1, bt1, W2, b2, g2, bt2, edge_index, batch):
  src = edge_index[0].astype(jnp.int32)
  dst = edge_index[1].astype(jnp.int32)
  nmain = NSUB * NCHUNK * CHUNK
  src_r = src[:nmain].reshape(NSUB, NCHUNK, CHUNK)
  dst_r = dst[:nmain].reshape(NSUB, NCHUNK, CHUNK)
  src_t = src[nmain:].reshape(NTAIL, 1, CHUNK)
  dst_t = dst[nmain:].reshape(NTAIL, 1, CHUNK)

  h2 = x.reshape(N, 2, DH).transpose(1, 0, 2)
  for l in range(NLAYERS):
    z2 = _sc_aggregate(h2, src_r, dst_r, src_t, dst_t)
    last = l == NLAYERS - 1
    h2 = _mlp_layer(
        z2,
        W1[l], b1[l].reshape(1, D), g1[l].reshape(1, D),
        bt1[l].reshape(1, D),
        W2[l], b2[l].reshape(1, D), g2[l].reshape(1, D),
        bt2[l].reshape(1, D),
        last)
  return (h2, batch)


# speedup vs baseline: 2.0698x; 1.2504x over previous
import functools

import jax
import jax.numpy as jnp
from jax import lax
from jax.experimental import pallas as pl
from jax.experimental.pallas import tpu as pltpu
from jax.experimental.pallas import tpu_sc as plsc

N = 10000
E = 160000
D = 256
DH = 128
NLAYERS = 3
NSUB = 16
CHUNK = 128
NCHUNK = 78
NTAIL = 2
EPW = NCHUNK * CHUNK
EPAD = NSUB * EPW
NROWS_ACC = N + 8
DUMMY_DST = N
SLAB = 632
SLAB_LAST = N - (NSUB - 1) * SLAB
BLK = 2000
NB = N // BLK
EPS = 1e-5


SEG = 26


def _sc_body(h_hbm, src_hbm, dst_hbm, st_hbm, dt_hbm, z_hbm, acc, sidx, didx,
             tidx_s, tidx_d, rows0, rows1, semA, semB):
  c = lax.axis_index("c")
  s = lax.axis_index("s")


  def slab_copy(src_ref, dst_ref):
    r0 = s * SLAB

    @pl.when(s < NSUB - 1)
    def _():
      pltpu.sync_copy(src_ref.at[pl.ds(r0, SLAB)],
                      dst_ref.at[pl.ds(r0, SLAB)])

    @pl.when(s == NSUB - 1)
    def _():
      pltpu.sync_copy(src_ref.at[pl.ds((NSUB - 1) * SLAB, SLAB_LAST)],
                      dst_ref.at[pl.ds((NSUB - 1) * SLAB, SLAB_LAST)])

  def run_half(h_half, z_half):
    slab_copy(h_half, acc)
    plsc.subcore_barrier()

    for seg in range(NCHUNK // SEG):
      pltpu.sync_copy(src_hbm.at[s, seg], sidx)
      pltpu.sync_copy(dst_hbm.at[s, seg], didx)
      pltpu.sync_copy(h_half.at[sidx.at[0]], rows0)

      @pl.loop(0, SEG - 2, step=2)
      def _(j):
        sA = pltpu.async_copy(rows0, acc.at[didx.at[j]], semA, add=True)
        pltpu.sync_copy(h_half.at[sidx.at[j + 1]], rows1)
        sA.wait()
        sB = pltpu.async_copy(rows1, acc.at[didx.at[j + 1]], semB, add=True)
        pltpu.sync_copy(h_half.at[sidx.at[j + 2]], rows0)
        sB.wait()

      sA = pltpu.async_copy(rows0, acc.at[didx.at[SEG - 2]], semA, add=True)
      pltpu.sync_copy(h_half.at[sidx.at[SEG - 1]], rows1)
      sA.wait()
      pltpu.sync_copy(rows1, acc.at[didx.at[SEG - 1]], add=True)

    @pl.when(s < NTAIL)
    def _():
      pltpu.sync_copy(st_hbm.at[s], tidx_s)
      pltpu.sync_copy(dt_hbm.at[s], tidx_d)
      pltpu.sync_copy(h_half.at[tidx_s.at[0]], rows0)
      pltpu.sync_copy(rows0, acc.at[tidx_d.at[0]], add=True)

    plsc.subcore_barrier()
    slab_copy(acc, z_half)

  @pl.when(c == 0)
  def _():
    run_half(h_hbm.at[0], z_hbm.at[0])

  @pl.when(c == 1)
  def _():
    run_half(h_hbm.at[1], z_hbm.at[1])


def _sc_aggregate(h2, src_r, dst_r, src_t, dst_t):
  mesh = plsc.VectorSubcoreMesh(core_axis_name="c", subcore_axis_name="s")
  kern = pl.kernel(
      _sc_body,
      out_type=jax.ShapeDtypeStruct((2, N, DH), jnp.float32),
      mesh=mesh,
      scratch_types=[
          pltpu.VMEM_SHARED((NROWS_ACC, DH), jnp.float32),
          pltpu.VMEM((SEG, CHUNK), jnp.int32),
          pltpu.VMEM((SEG, CHUNK), jnp.int32),
          pltpu.VMEM((1, CHUNK), jnp.int32),
          pltpu.VMEM((1, CHUNK), jnp.int32),
          pltpu.VMEM((CHUNK, DH), jnp.float32),
          pltpu.VMEM((CHUNK, DH), jnp.float32),
          pltpu.SemaphoreType.DMA,
          pltpu.SemaphoreType.DMA,
      ],
  )
  return kern(h2, src_r, dst_r, src_t, dst_t)


def _mlp_body(last, z, w1, b1, g1, bt1, w2, b2, g2, bt2, out, u):
  w1m = w1[...]
  w2m = w2[...]
  b1v = b1[...]
  b2v = b2[...]
  zero = jnp.zeros((1, D), jnp.float32)

  def p1(i, carry):
    s0, s1 = carry
    zL = z[0, pl.ds(i * BLK, BLK), :]
    zR = z[1, pl.ds(i * BLK, BLK), :]
    ub = (jnp.dot(zL, w1m[:DH, :], preferred_element_type=jnp.float32)
          + jnp.dot(zR, w1m[DH:, :], preferred_element_type=jnp.float32)
          + b1v)
    u[pl.ds(i * BLK, BLK), :] = ub
    return (s0 + jnp.sum(ub, axis=0, keepdims=True),
            s1 + jnp.sum(ub * ub, axis=0, keepdims=True))

  s0, s1 = lax.fori_loop(0, NB, p1, (zero, zero))
  m1 = s0 / N
  v1 = s1 / N - m1 * m1
  sc1 = g1[...] * lax.rsqrt(v1 + EPS)
  sh1 = bt1[...] - m1 * sc1

  def p2(i, carry):
    s0, s1 = carry
    ub = u[pl.ds(i * BLK, BLK), :]
    r = jnp.maximum(ub * sc1 + sh1, 0.0)
    sb = jnp.dot(r, w2m, preferred_element_type=jnp.float32) + b2v
    u[pl.ds(i * BLK, BLK), :] = sb
    return (s0 + jnp.sum(sb, axis=0, keepdims=True),
            s1 + jnp.sum(sb * sb, axis=0, keepdims=True))

  s0, s1 = lax.fori_loop(0, NB, p2, (zero, zero))
  m2 = s0 / N
  v2 = s1 / N - m2 * m2
  sc2 = g2[...] * lax.rsqrt(v2 + EPS)
  sh2 = bt2[...] - m2 * sc2

  def p3(i, _):
    sb = u[pl.ds(i * BLK, BLK), :]
    h = sb * sc2 + sh2
    if not last:
      h = jnp.maximum(h, 0.0)
      out[0, pl.ds(i * BLK, BLK), :] = h[:, :DH]
      out[1, pl.ds(i * BLK, BLK), :] = h[:, DH:]
    else:
      out[pl.ds(i * BLK, BLK), :] = h
    return 0

  lax.fori_loop(0, NB, p3, 0)


def _mlp_layer(z2, w1, b1, g1, bt1, w2, b2, g2, bt2, last):
  out_shape = (jax.ShapeDtypeStruct((N, D), jnp.float32) if last
               else jax.ShapeDtypeStruct((2, N, DH), jnp.float32))
  return pl.pallas_call(
      functools.partial(_mlp_body, last),
      out_shape=out_shape,
      scratch_shapes=[pltpu.VMEM((N, D), jnp.float32)],
  )(z2, w1, b1, g1, bt1, w2, b2, g2, bt2)


def kernel(x, W1, b1, g1, bt1, W2, b2, g2, bt2, edge_index, batch):
  src = edge_index[0].astype(jnp.int32)
  dst = edge_index[1].astype(jnp.int32)
  nmain = NSUB * NCHUNK * CHUNK
  src_r = src[:nmain].reshape(NSUB, NCHUNK // SEG, SEG, CHUNK)
  dst_r = dst[:nmain].reshape(NSUB, NCHUNK // SEG, SEG, CHUNK)
  src_t = src[nmain:].reshape(NTAIL, 1, CHUNK)
  dst_t = dst[nmain:].reshape(NTAIL, 1, CHUNK)

  h2 = x.reshape(N, 2, DH).transpose(1, 0, 2)
  for l in range(NLAYERS):
    z2 = _sc_aggregate(h2, src_r, dst_r, src_t, dst_t)
    last = l == NLAYERS - 1
    h2 = _mlp_layer(
        z2,
        W1[l], b1[l].reshape(1, D), g1[l].reshape(1, D),
        bt1[l].reshape(1, D),
        W2[l], b2[l].reshape(1, D), g2[l].reshape(1, D),
        bt2[l].reshape(1, D),
        last)
  return (h2, batch)


# speedup vs baseline: 2.1018x; 1.0154x over previous
import functools

import jax
import jax.numpy as jnp
from jax import lax
from jax.experimental import pallas as pl
from jax.experimental.pallas import tpu as pltpu
from jax.experimental.pallas import tpu_sc as plsc

N = 10000
E = 160000
D = 256
DH = 128
NLAYERS = 3
NSUB = 16
CHUNK = 128
NCHUNK = 78
NTAIL = 2
EPW = NCHUNK * CHUNK
EPAD = NSUB * EPW
NROWS_ACC = N + 8
DUMMY_DST = N
SLAB = 632
SLAB_LAST = N - (NSUB - 1) * SLAB
BLK = 2000
NB = N // BLK
EPS = 1e-5


SEG = 26


def _sc_body(h_hbm, src_hbm, dst_hbm, st_hbm, dt_hbm, z_hbm, acc, sidx, didx,
             tidx_s, tidx_d, rows0, rows1, semA, semB):
  c = lax.axis_index("c")
  s = lax.axis_index("s")


  def slab_copy(src_ref, dst_ref):
    r0 = s * SLAB

    @pl.when(s < NSUB - 1)
    def _():
      pltpu.sync_copy(src_ref.at[pl.ds(r0, SLAB)],
                      dst_ref.at[pl.ds(r0, SLAB)])

    @pl.when(s == NSUB - 1)
    def _():
      pltpu.sync_copy(src_ref.at[pl.ds((NSUB - 1) * SLAB, SLAB_LAST)],
                      dst_ref.at[pl.ds((NSUB - 1) * SLAB, SLAB_LAST)])

  def run_half(h_half, z_half):
    slab_copy(h_half, acc)
    plsc.subcore_barrier()

    for seg in range(NCHUNK // SEG):
      pltpu.sync_copy(src_hbm.at[s, seg], sidx)
      pltpu.sync_copy(dst_hbm.at[s, seg], didx)
      pltpu.sync_copy(h_half.at[sidx.at[0]], rows0)

      @pl.loop(0, SEG - 2, step=2)
      def _(j):
        sA = pltpu.async_copy(rows0, acc.at[didx.at[j]], semA, add=True)
        pltpu.sync_copy(h_half.at[sidx.at[j + 1]], rows1)
        sA.wait()
        sB = pltpu.async_copy(rows1, acc.at[didx.at[j + 1]], semB, add=True)
        pltpu.sync_copy(h_half.at[sidx.at[j + 2]], rows0)
        sB.wait()

      sA = pltpu.async_copy(rows0, acc.at[didx.at[SEG - 2]], semA, add=True)
      pltpu.sync_copy(h_half.at[sidx.at[SEG - 1]], rows1)
      sA.wait()
      pltpu.sync_copy(rows1, acc.at[didx.at[SEG - 1]], add=True)

    @pl.when(s < NTAIL)
    def _():
      pltpu.sync_copy(st_hbm.at[s], tidx_s)
      pltpu.sync_copy(dt_hbm.at[s], tidx_d)
      pltpu.sync_copy(h_half.at[tidx_s.at[0]], rows0)
      pltpu.sync_copy(rows0, acc.at[tidx_d.at[0]], add=True)

    plsc.subcore_barrier()
    slab_copy(acc, z_half)

  @pl.when(c == 0)
  def _():
    run_half(h_hbm.at[0], z_hbm.at[0])

  @pl.when(c == 1)
  def _():
    run_half(h_hbm.at[1], z_hbm.at[1])


def _sc_aggregate(h2, src_r, dst_r, src_t, dst_t):
  mesh = plsc.VectorSubcoreMesh(core_axis_name="c", subcore_axis_name="s")
  kern = pl.kernel(
      _sc_body,
      out_type=jax.ShapeDtypeStruct((2, N, DH), jnp.float32),
      mesh=mesh,
      scratch_types=[
          pltpu.VMEM_SHARED((NROWS_ACC, DH), jnp.float32),
          pltpu.VMEM((SEG, CHUNK), jnp.int32),
          pltpu.VMEM((SEG, CHUNK), jnp.int32),
          pltpu.VMEM((1, CHUNK), jnp.int32),
          pltpu.VMEM((1, CHUNK), jnp.int32),
          pltpu.VMEM((CHUNK, DH), jnp.float32),
          pltpu.VMEM((CHUNK, DH), jnp.float32),
          pltpu.SemaphoreType.DMA,
          pltpu.SemaphoreType.DMA,
      ],
  )
  return kern(h2, src_r, dst_r, src_t, dst_t)


def _mlp_body(last, z, w1, b1, g1, bt1, w2, b2, g2, bt2, out, u):
  w1m = w1[...]
  w2m = w2[...]
  b1v = b1[...]
  b2v = b2[...]
  zero = jnp.zeros((1, D), jnp.float32)

  def p1(i, carry):
    s0, s1 = carry
    zL = z[0, pl.ds(i * BLK, BLK), :]
    zR = z[1, pl.ds(i * BLK, BLK), :]
    ub = (jnp.dot(zL, w1m[:DH, :], preferred_element_type=jnp.float32)
          + jnp.dot(zR, w1m[DH:, :], preferred_element_type=jnp.float32)
          + b1v)
    u[pl.ds(i * BLK, BLK), :] = ub
    return (s0 + jnp.sum(ub, axis=0, keepdims=True),
            s1 + jnp.sum(ub * ub, axis=0, keepdims=True))

  s0, s1 = lax.fori_loop(0, NB, p1, (zero, zero))
  m1 = s0 / N
  v1 = s1 / N - m1 * m1
  sc1 = g1[...] * lax.rsqrt(v1 + EPS)
  sh1 = bt1[...] - m1 * sc1

  def p2(i, carry):
    s0, s1 = carry
    ub = u[pl.ds(i * BLK, BLK), :]
    r = jnp.maximum(ub * sc1 + sh1, 0.0)
    sb = jnp.dot(r, w2m, preferred_element_type=jnp.float32) + b2v
    u[pl.ds(i * BLK, BLK), :] = sb
    return (s0 + jnp.sum(sb, axis=0, keepdims=True),
            s1 + jnp.sum(sb * sb, axis=0, keepdims=True))

  s0, s1 = lax.fori_loop(0, NB, p2, (zero, zero))
  m2 = s0 / N
  v2 = s1 / N - m2 * m2
  sc2 = g2[...] * lax.rsqrt(v2 + EPS)
  sh2 = bt2[...] - m2 * sc2

  def p3(i, _):
    sb = u[pl.ds(i * BLK, BLK), :]
    h = sb * sc2 + sh2
    if not last:
      h = jnp.maximum(h, 0.0)
      out[0, pl.ds(i * BLK, BLK), :] = h[:, :DH]
      out[1, pl.ds(i * BLK, BLK), :] = h[:, DH:]
    else:
      out[pl.ds(i * BLK, BLK), :] = h
    return 0

  lax.fori_loop(0, NB, p3, 0)


def _mlp_layer(z2, w1, b1, g1, bt1, w2, b2, g2, bt2, last):
  out_shape = (jax.ShapeDtypeStruct((N, D), jnp.float32) if last
               else jax.ShapeDtypeStruct((2, N, DH), jnp.float32))
  return pl.pallas_call(
      functools.partial(_mlp_body, last),
      out_shape=out_shape,
      scratch_shapes=[pltpu.VMEM((N, D), jnp.float32)],
  )(z2, w1, b1, g1, bt1, w2, b2, g2, bt2)


def _split_body(x_ref, out_ref):
  out_ref[0, :, :] = x_ref[:, :DH]
  out_ref[1, :, :] = x_ref[:, DH:]


def _split_halves(x):
  return pl.pallas_call(
      _split_body,
      out_shape=jax.ShapeDtypeStruct((2, N, DH), jnp.float32),
  )(x)


def kernel(x, W1, b1, g1, bt1, W2, b2, g2, bt2, edge_index, batch):
  src = edge_index[0].astype(jnp.int32)
  dst = edge_index[1].astype(jnp.int32)
  nmain = NSUB * NCHUNK * CHUNK
  src_r = src[:nmain].reshape(NSUB, NCHUNK // SEG, SEG, CHUNK)
  dst_r = dst[:nmain].reshape(NSUB, NCHUNK // SEG, SEG, CHUNK)
  src_t = src[nmain:].reshape(NTAIL, 1, CHUNK)
  dst_t = dst[nmain:].reshape(NTAIL, 1, CHUNK)

  h2 = _split_halves(x)
  for l in range(NLAYERS):
    z2 = _sc_aggregate(h2, src_r, dst_r, src_t, dst_t)
    last = l == NLAYERS - 1
    h2 = _mlp_layer(
        z2,
        W1[l], b1[l].reshape(1, D), g1[l].reshape(1, D),
        bt1[l].reshape(1, D),
        W2[l], b2[l].reshape(1, D), g2[l].reshape(1, D),
        bt2[l].reshape(1, D),
        last)
  return (h2, batch)
